# baseline (speedup 1.0000x reference)
import functools

import jax
import jax.numpy as jnp
from jax.experimental import pallas as pl
from jax.experimental.pallas import tpu as pltpu

_EPS = 1e-5
_NEG = -1e30


def _body(x_ref, mpt_ref, mrow_ref, a_in_ref,
          wpreT_ref, gpre_ref, bpre_ref,
          w1T_ref, g1_ref, b1_ref,
          w2T_ref, g2_ref, b2_ref,
          wo1T_ref, bo1_ref, wo2T_ref, bo2_ref,
          out_ref, a_out_ref, stat, segmax,
          *, G, NPAD, H):
    ph = pl.program_id(0)
    i = pl.program_id(1)
    nb = pl.num_programs(1)
    R = G * NPAD

    @pl.when(jnp.logical_and(ph == 0, i == 0))
    def _init():
        stat[...] = jnp.zeros_like(stat)

    def accum(a):
        mT = mrow_ref[...].reshape(1, R)
        stat[0:1] += jnp.dot(mT, a, preferred_element_type=jnp.float32)
        stat[1:2] += jnp.dot(mT, a * a, preferred_element_type=jnp.float32)

    def finalize(g_ref, b_ref, srow):
        cnt = jnp.maximum(stat[14:15], 1.0)
        mean = stat[0:1] / cnt
        var = stat[1:2] / cnt - mean * mean
        s = g_ref[...] / jnp.sqrt(var + _EPS)
        t = b_ref[...] - mean * s
        stat[srow:srow + 1] = s
        stat[srow + 1:srow + 2] = t
        stat[0:2] = jnp.zeros((2, H), jnp.float32)

    def a1_fn():
        return jnp.dot(x_ref[...], wpreT_ref[...],
                       preferred_element_type=jnp.float32)

    @pl.when(ph == 0)
    def _p0():
        a1 = a1_fn()
        accum(a1)
        stat[14:15] += jnp.sum(mpt_ref[...])

    @pl.when(jnp.logical_and(ph == 0, i == nb - 1))
    def _f0():
        finalize(gpre_ref, bpre_ref, 8)

    @pl.when(ph == 1)
    def _p1():
        m = mpt_ref[...]
        a1 = a1_fn()
        feat = jnp.maximum(a1 * stat[8:9] + stat[9:10], 0.0) * m
        pooled = jnp.max(feat.reshape(G, NPAD, H), axis=1)
        pc = jnp.dot(pooled, w1T_ref[H:2 * H, :],
                     preferred_element_type=jnp.float32)
        pc3 = jnp.broadcast_to(pc[:, None, :], (G, NPAD, H)).reshape(R, H)
        a2 = jnp.dot(feat, w1T_ref[0:H, :],
                     preferred_element_type=jnp.float32) + pc3
        accum(a2)
        a_out_ref[...] = a2

    @pl.when(jnp.logical_and(ph == 1, i == nb - 1))
    def _f1():
        finalize(g1_ref, b1_ref, 10)

    @pl.when(ph == 2)
    def _p2():
        m = mpt_ref[...]
        a2 = a_in_ref[...]
        h2 = jnp.maximum(a2 * stat[10:11] + stat[11:12], 0.0) * m
        a3 = jnp.dot(h2, w2T_ref[...], preferred_element_type=jnp.float32)
        accum(a3)
        z = jnp.where(m > 0.0, a3, _NEG)
        segmax[pl.ds(i * G, G), :] = jnp.max(z.reshape(G, NPAD, H), axis=1)

    @pl.when(jnp.logical_and(ph == 2, i == nb - 1))
    def _f2():
        finalize(g2_ref, b2_ref, 12)

    @pl.when(ph == 3)
    def _p3():
        sm = segmax[pl.ds(i * G, G), :]
        buf = jnp.maximum(sm * stat[12:13] + stat[13:14], 0.0)
        o1 = jnp.maximum(
            jnp.dot(buf, wo1T_ref[...], preferred_element_type=jnp.float32)
            + bo1_ref[...], 0.0)
        o = jnp.dot(o1, wo2T_ref[...],
                    preferred_element_type=jnp.float32) + bo2_ref[...]
        valid = sm[:, 0:1] > (0.5 * _NEG)
        out_ref[...] = o * valid.astype(jnp.float32)


def kernel(polylines, polylines_mask, W_pre, g_pre, b_pre,
           W1, g1, b1, W2, g2, b2, Wo1, bo1, Wo2, bo2):
    B, P, N, C = polylines.shape
    H = W_pre.shape[0]
    O = Wo2.shape[0]
    BP = B * P
    NPAD = ((N + 7) // 8) * 8
    G = 256
    NB = BP // G
    R = G * NPAD

    xp = jnp.pad(polylines.reshape(BP, N, C),
                 ((0, 0), (0, NPAD - N), (0, 0))).reshape(BP * NPAD, C)
    mf = jnp.pad(polylines_mask.astype(jnp.float32).reshape(BP, N),
                 ((0, 0), (0, NPAD - N)))
    mpt = mf.reshape(BP * NPAD, 1)
    mrow = mf.reshape(NB, 1, R)
    a_buf = jnp.zeros((BP * NPAD, H), jnp.float32)

    row = lambda v: v.reshape(1, -1)

    def x_idx(ph, i):
        return (jnp.where(ph < 2, i, 0), 0)

    def pts_idx(ph, i):
        return (jnp.where(ph < 3, i, 0), 0)

    def mrow_idx(ph, i):
        return (jnp.where(ph < 3, i, 0), 0, 0)

    def a_in_idx(ph, i):
        return (jnp.where(ph == 2, i, 1), 0)

    def a_out_idx(ph, i):
        return (jnp.where(ph == 1, i, 0), 0)

    def poly_idx(ph, i):
        return (jnp.where(ph == 3, i, 0), 0)

    full = lambda shape: pl.BlockSpec(shape, lambda ph, i: (0, 0))

    body = functools.partial(_body, G=G, NPAD=NPAD, H=H)

    out, _ = pl.pallas_call(
        body,
        grid=(4, NB),
        in_specs=[
            pl.BlockSpec((R, C), x_idx),
            pl.BlockSpec((R, 1), pts_idx),
            pl.BlockSpec((1, 1, R), mrow_idx),
            pl.BlockSpec((R, H), a_in_idx),
            full((C, H)), full((1, H)), full((1, H)),
            full((2 * H, H)), full((1, H)), full((1, H)),
            full((H, H)), full((1, H)), full((1, H)),
            full((H, H)), full((1, H)), full((H, O)), full((1, O)),
        ],
        out_specs=[
            pl.BlockSpec((G, O), poly_idx),
            pl.BlockSpec((R, H), a_out_idx),
        ],
        out_shape=[
            jax.ShapeDtypeStruct((BP, O), jnp.float32),
            jax.ShapeDtypeStruct((BP * NPAD, H), jnp.float32),
        ],
        input_output_aliases={3: 1},
        scratch_shapes=[
            pltpu.VMEM((16, H), jnp.float32),
            pltpu.VMEM((BP, H), jnp.float32),
        ],
    )(xp, mpt, mrow, a_buf,
      W_pre.T, row(g_pre), row(b_pre),
      W1.T, row(g1), row(b1),
      W2.T, row(g2), row(b2),
      Wo1.T, row(bo1), Wo2.T, row(bo2))
    return out.reshape(B, P, O)


# speedup vs baseline: 1.0817x; 1.0817x over previous
import functools

import jax
import jax.numpy as jnp
from jax.experimental import pallas as pl
from jax.experimental.pallas import tpu as pltpu

_EPS = 1e-5
_NEG = -1e30


def _body(x_ref, mpt_ref, mrow_ref, a_in_ref,
          wpreT_ref, gpre_ref, bpre_ref,
          w1T_ref, g1_ref, b1_ref,
          w2T_ref, g2_ref, b2_ref,
          wo1T_ref, bo1_ref, wo2T_ref, bo2_ref,
          out_ref, a_out_ref, stat, segmax,
          *, G, NPAD, H):
    ph = pl.program_id(0)
    i = pl.program_id(1)
    nb = pl.num_programs(1)
    R = G * NPAD

    @pl.when(jnp.logical_and(ph == 0, i == 0))
    def _init():
        stat[...] = jnp.zeros_like(stat)

    def accum(a):
        mT = mrow_ref[...].reshape(1, R)
        stat[0:1] += jnp.dot(mT, a, preferred_element_type=jnp.float32)
        stat[1:2] += jnp.dot(mT, a * a, preferred_element_type=jnp.float32)

    def finalize(g_ref, b_ref, srow):
        cnt = jnp.maximum(stat[14:15], 1.0)
        mean = stat[0:1] / cnt
        var = stat[1:2] / cnt - mean * mean
        s = g_ref[...] / jnp.sqrt(var + _EPS)
        t = b_ref[...] - mean * s
        stat[srow:srow + 1] = s
        stat[srow + 1:srow + 2] = t
        stat[0:2] = jnp.zeros((2, H), jnp.float32)

    def a1_fn():
        return jnp.dot(x_ref[...], wpreT_ref[...],
                       preferred_element_type=jnp.float32)

    @pl.when(ph == 0)
    def _p0():
        a1 = a1_fn()
        accum(a1)
        stat[14:15] += jnp.sum(mpt_ref[...])

    @pl.when(jnp.logical_and(ph == 0, i == nb - 1))
    def _f0():
        finalize(gpre_ref, bpre_ref, 8)

    @pl.when(ph == 1)
    def _p1():
        m = mpt_ref[...]
        a1 = a1_fn()
        feat = jnp.maximum(a1 * stat[8:9] + stat[9:10], 0.0) * m
        pooled = jnp.max(feat.reshape(G, NPAD, H), axis=1)
        pc = jnp.dot(pooled, w1T_ref[H:2 * H, :],
                     preferred_element_type=jnp.float32)
        pc3 = jnp.broadcast_to(pc[:, None, :], (G, NPAD, H)).reshape(R, H)
        a2 = jnp.dot(feat, w1T_ref[0:H, :],
                     preferred_element_type=jnp.float32) + pc3
        accum(a2)
        a_out_ref[...] = a2.astype(jnp.bfloat16)

    @pl.when(jnp.logical_and(ph == 1, i == nb - 1))
    def _f1():
        finalize(g1_ref, b1_ref, 10)

    @pl.when(ph == 2)
    def _p2():
        m = mpt_ref[...]
        a2 = a_in_ref[...].astype(jnp.float32)
        h2 = jnp.maximum(a2 * stat[10:11] + stat[11:12], 0.0) * m
        a3 = jnp.dot(h2, w2T_ref[...], preferred_element_type=jnp.float32)
        accum(a3)
        z = jnp.where(m > 0.0, a3, _NEG)
        segmax[pl.ds(i * G, G), :] = jnp.max(z.reshape(G, NPAD, H), axis=1)

    @pl.when(jnp.logical_and(ph == 2, i == nb - 1))
    def _f2():
        finalize(g2_ref, b2_ref, 12)

    @pl.when(ph == 3)
    def _p3():
        sm = segmax[pl.ds(i * G, G), :]
        buf = jnp.maximum(sm * stat[12:13] + stat[13:14], 0.0)
        o1 = jnp.maximum(
            jnp.dot(buf, wo1T_ref[...], preferred_element_type=jnp.float32)
            + bo1_ref[...], 0.0)
        o = jnp.dot(o1, wo2T_ref[...],
                    preferred_element_type=jnp.float32) + bo2_ref[...]
        valid = sm[:, 0:1] > (0.5 * _NEG)
        out_ref[...] = o * valid.astype(jnp.float32)


def kernel(polylines, polylines_mask, W_pre, g_pre, b_pre,
           W1, g1, b1, W2, g2, b2, Wo1, bo1, Wo2, bo2):
    B, P, N, C = polylines.shape
    H = W_pre.shape[0]
    O = Wo2.shape[0]
    BP = B * P
    NPAD = ((N + 7) // 8) * 8
    G = 256
    NB = BP // G
    R = G * NPAD

    xp = jnp.pad(polylines.reshape(BP, N, C),
                 ((0, 0), (0, NPAD - N), (0, 0))).reshape(BP * NPAD, C)
    mf = jnp.pad(polylines_mask.astype(jnp.float32).reshape(BP, N),
                 ((0, 0), (0, NPAD - N)))
    mpt = mf.reshape(BP * NPAD, 1)
    mrow = mf.reshape(NB, 1, R)
    a_buf = jnp.zeros((BP * NPAD, H), jnp.bfloat16)

    row = lambda v: v.reshape(1, -1)

    def x_idx(ph, i):
        return (jnp.where(ph < 2, i, 0), 0)

    def pts_idx(ph, i):
        return (jnp.where(ph < 3, i, 0), 0)

    def mrow_idx(ph, i):
        return (jnp.where(ph < 3, i, 0), 0, 0)

    def a_in_idx(ph, i):
        return (jnp.where(ph == 2, i, 1), 0)

    def a_out_idx(ph, i):
        return (jnp.where(ph == 1, i, 0), 0)

    def poly_idx(ph, i):
        return (jnp.where(ph == 3, i, 0), 0)

    full = lambda shape: pl.BlockSpec(shape, lambda ph, i: (0, 0))

    body = functools.partial(_body, G=G, NPAD=NPAD, H=H)

    out, _ = pl.pallas_call(
        body,
        grid=(4, NB),
        in_specs=[
            pl.BlockSpec((R, C), x_idx),
            pl.BlockSpec((R, 1), pts_idx),
            pl.BlockSpec((1, 1, R), mrow_idx),
            pl.BlockSpec((R, H), a_in_idx),
            full((C, H)), full((1, H)), full((1, H)),
            full((2 * H, H)), full((1, H)), full((1, H)),
            full((H, H)), full((1, H)), full((1, H)),
            full((H, H)), full((1, H)), full((H, O)), full((1, O)),
        ],
        out_specs=[
            pl.BlockSpec((G, O), poly_idx),
            pl.BlockSpec((R, H), a_out_idx),
        ],
        out_shape=[
            jax.ShapeDtypeStruct((BP, O), jnp.float32),
            jax.ShapeDtypeStruct((BP * NPAD, H), jnp.bfloat16),
        ],
        input_output_aliases={3: 1},
        scratch_shapes=[
            pltpu.VMEM((16, H), jnp.float32),
            pltpu.VMEM((BP, H), jnp.float32),
        ],
    )(xp, mpt, mrow, a_buf,
      W_pre.T, row(g_pre), row(b_pre),
      W1.T, row(g1), row(b1),
      W2.T, row(g2), row(b2),
      Wo1.T, row(bo1), Wo2.T, row(bo2))
    return out.reshape(B, P, O)


# speedup vs baseline: 1.1515x; 1.0646x over previous
import functools

import jax
import jax.numpy as jnp
from jax.experimental import pallas as pl
from jax.experimental.pallas import tpu as pltpu

_EPS = 1e-5
_NEG = -1e30


def _body(x_ref, mpt_ref, a_in_ref,
          wpreT_ref, gpre_ref, bpre_ref,
          w1T_ref, g1_ref, b1_ref,
          w2T_ref, g2_ref, b2_ref,
          wo1T_ref, bo1_ref, wo2T_ref, bo2_ref,
          out_ref, a_out_ref, stat, segmax,
          *, G, NPAD, H, C9):
    ph = pl.program_id(0)
    i = pl.program_id(1)
    nb = pl.num_programs(1)
    R = G * NPAD

    @pl.when(jnp.logical_and(ph == 0, i == 0))
    def _init():
        stat[...] = jnp.zeros_like(stat)

    def accum(a, m):
        am = a * m
        stat[0:1] += jnp.sum(am, axis=0, keepdims=True)
        stat[1:2] += jnp.sum(am * a, axis=0, keepdims=True)

    def finalize(g_ref, b_ref, srow):
        cnt = jnp.maximum(stat[14:15], 1.0)
        mean = stat[0:1] / cnt
        var = stat[1:2] / cnt - mean * mean
        s = g_ref[...] / jnp.sqrt(var + _EPS)
        t = b_ref[...] - mean * s
        stat[srow:srow + 1] = s
        stat[srow + 1:srow + 2] = t
        stat[0:2] = jnp.zeros((2, H), jnp.float32)

    def a1_fn():
        return jnp.dot(x_ref[...], wpreT_ref[...],
                       preferred_element_type=jnp.float32)

    @pl.when(ph == 0)
    def _p0():
        m = mpt_ref[...].astype(jnp.float32)
        x = x_ref[...]
        xm = x * m
        stat[16:16 + C9, 0:C9] += jax.lax.dot_general(
            xm, x, (((0,), (0,)), ((), ())),
            preferred_element_type=jnp.float32)
        stat[15:16, 0:C9] += jnp.sum(xm, axis=0, keepdims=True)
        stat[14:15] += jnp.sum(m)

    @pl.when(jnp.logical_and(ph == 0, i == nb - 1))
    def _f0():
        cnt = jnp.maximum(stat[14:15], 1.0)
        wpre = wpreT_ref[...]
        mean = jnp.dot(stat[15:16, 0:C9], wpre,
                       preferred_element_type=jnp.float32) / cnt
        t_w = jnp.dot(stat[16:16 + C9, 0:C9], wpre,
                      preferred_element_type=jnp.float32)
        e2 = jnp.sum(t_w * wpre, axis=0, keepdims=True) / cnt
        var = e2 - mean * mean
        s = gpre_ref[...] / jnp.sqrt(var + _EPS)
        stat[8:9] = s
        stat[9:10] = bpre_ref[...] - mean * s

    @pl.when(ph == 1)
    def _p1():
        m = mpt_ref[...].astype(jnp.float32)
        a1 = a1_fn()
        feat = jnp.maximum(a1 * stat[8:9] + stat[9:10], 0.0) * m
        pooled = jnp.max(feat.reshape(G, NPAD, H), axis=1)
        pc = jnp.dot(pooled, w1T_ref[H:2 * H, :],
                     preferred_element_type=jnp.float32)
        pc3 = jnp.broadcast_to(pc[:, None, :], (G, NPAD, H)).reshape(R, H)
        a2 = jnp.dot(feat, w1T_ref[0:H, :],
                     preferred_element_type=jnp.float32) + pc3
        accum(a2, m)
        a_out_ref[...] = a2.astype(jnp.bfloat16)

    @pl.when(jnp.logical_and(ph == 1, i == nb - 1))
    def _f1():
        finalize(g1_ref, b1_ref, 10)

    @pl.when(ph == 2)
    def _p2():
        m = mpt_ref[...].astype(jnp.float32)
        a2 = a_in_ref[...].astype(jnp.float32)
        h2 = jnp.maximum(a2 * stat[10:11] + stat[11:12], 0.0) * m
        a3 = jnp.dot(h2, w2T_ref[...], preferred_element_type=jnp.float32)
        accum(a3, m)
        z = jnp.where(m > 0.0, a3, _NEG)
        segmax[pl.ds(i * G, G), :] = jnp.max(z.reshape(G, NPAD, H), axis=1)

    @pl.when(jnp.logical_and(ph == 2, i == nb - 1))
    def _f2():
        finalize(g2_ref, b2_ref, 12)

    @pl.when(ph == 3)
    def _p3():
        sm = segmax[pl.ds(i * G, G), :]
        buf = jnp.maximum(sm * stat[12:13] + stat[13:14], 0.0)
        o1 = jnp.maximum(
            jnp.dot(buf, wo1T_ref[...], preferred_element_type=jnp.float32)
            + bo1_ref[...], 0.0)
        o = jnp.dot(o1, wo2T_ref[...],
                    preferred_element_type=jnp.float32) + bo2_ref[...]
        valid = sm[:, 0:1] > (0.5 * _NEG)
        out_ref[...] = o * valid.astype(jnp.float32)


def kernel(polylines, polylines_mask, W_pre, g_pre, b_pre,
           W1, g1, b1, W2, g2, b2, Wo1, bo1, Wo2, bo2):
    B, P, N, C = polylines.shape
    H = W_pre.shape[0]
    O = Wo2.shape[0]
    BP = B * P
    NPAD = ((N + 7) // 8) * 8
    G = 256
    NB = BP // G
    R = G * NPAD

    xp = jnp.pad(polylines.reshape(BP, N, C),
                 ((0, 0), (0, NPAD - N), (0, 0))).reshape(BP * NPAD, C)
    mpt = jnp.pad(polylines_mask.astype(jnp.bfloat16).reshape(BP, N),
                  ((0, 0), (0, NPAD - N))).reshape(BP * NPAD, 1)
    a_buf = jnp.zeros((BP * NPAD, H), jnp.bfloat16)

    row = lambda v: v.reshape(1, -1)

    def x_idx(ph, i):
        return (jnp.where(ph < 2, i, 0), 0)

    def pts_idx(ph, i):
        return (jnp.where(ph < 3, i, 0), 0)

    def a_in_idx(ph, i):
        return (jnp.where(ph == 2, i, 1), 0)

    def a_out_idx(ph, i):
        return (jnp.where(ph == 1, i, 0), 0)

    def poly_idx(ph, i):
        return (jnp.where(ph == 3, i, 0), 0)

    full = lambda shape: pl.BlockSpec(shape, lambda ph, i: (0, 0))

    body = functools.partial(_body, G=G, NPAD=NPAD, H=H, C9=C)

    out, _ = pl.pallas_call(
        body,
        grid=(4, NB),
        in_specs=[
            pl.BlockSpec((R, C), x_idx),
            pl.BlockSpec((R, 1), pts_idx),
            pl.BlockSpec((R, H), a_in_idx),
            full((C, H)), full((1, H)), full((1, H)),
            full((2 * H, H)), full((1, H)), full((1, H)),
            full((H, H)), full((1, H)), full((1, H)),
            full((H, H)), full((1, H)), full((H, O)), full((1, O)),
        ],
        out_specs=[
            pl.BlockSpec((G, O), poly_idx),
            pl.BlockSpec((R, H), a_out_idx),
        ],
        out_shape=[
            jax.ShapeDtypeStruct((BP, O), jnp.float32),
            jax.ShapeDtypeStruct((BP * NPAD, H), jnp.bfloat16),
        ],
        input_output_aliases={2: 1},
        scratch_shapes=[
            pltpu.VMEM((32, H), jnp.float32),
            pltpu.VMEM((BP, H), jnp.float32),
        ],
    )(xp, mpt, a_buf,
      W_pre.T, row(g_pre), row(b_pre),
      W1.T, row(g1), row(b1),
      W2.T, row(g2), row(b2),
      Wo1.T, row(bo1), Wo2.T, row(bo2))
    return out.reshape(B, P, O)


# speedup vs baseline: 1.2156x; 1.0556x over previous
import functools

import jax
import jax.numpy as jnp
from jax.experimental import pallas as pl
from jax.experimental.pallas import tpu as pltpu

_EPS = 1e-5
_NEG = -1e30


def _body(x_ref, mpt_ref, a_in_ref,
          wpreT_ref, gpre_ref, bpre_ref,
          w1T_ref, g1_ref, b1_ref,
          w2T_ref, g2_ref, b2_ref,
          wo1T_ref, bo1_ref, wo2T_ref, bo2_ref,
          out_ref, a_out_ref, stat, segmax,
          *, G, NPAD, H, C9):
    ph = pl.program_id(0)
    i = pl.program_id(1)
    nb = pl.num_programs(1)
    R = G * NPAD

    @pl.when(jnp.logical_and(ph == 0, i == 0))
    def _init():
        stat[...] = jnp.zeros_like(stat)

    def accum(a, m):
        am = a * m
        stat[0:1] += jnp.sum(am, axis=0, keepdims=True)
        stat[1:2] += jnp.sum(am * a, axis=0, keepdims=True)

    def finalize(g_ref, b_ref, srow):
        cnt = jnp.maximum(stat[14:15], 1.0)
        mean = stat[0:1] / cnt
        var = stat[1:2] / cnt - mean * mean
        s = g_ref[...] / jnp.sqrt(var + _EPS)
        t = b_ref[...] - mean * s
        stat[srow:srow + 1] = s
        stat[srow + 1:srow + 2] = t
        stat[0:2] = jnp.zeros((2, H), jnp.float32)

    def a1_fn():
        return jnp.dot(x_ref[...], wpreT_ref[...],
                       preferred_element_type=jnp.float32)

    @pl.when(ph == 0)
    def _p0():
        m = mpt_ref[...].astype(jnp.float32)
        x = x_ref[...]
        xm = x * m
        stat[16:16 + C9, 0:C9] += jax.lax.dot_general(
            xm, x, (((0,), (0,)), ((), ())),
            preferred_element_type=jnp.float32)
        stat[15:16, 0:C9] += jnp.sum(xm, axis=0, keepdims=True)
        stat[14:15] += jnp.sum(m)

    @pl.when(jnp.logical_and(ph == 0, i == nb - 1))
    def _f0():
        cnt = jnp.maximum(stat[14:15], 1.0)
        wpre = wpreT_ref[...]
        mean = jnp.dot(stat[15:16, 0:C9], wpre,
                       preferred_element_type=jnp.float32) / cnt
        t_w = jnp.dot(stat[16:16 + C9, 0:C9], wpre,
                      preferred_element_type=jnp.float32)
        e2 = jnp.sum(t_w * wpre, axis=0, keepdims=True) / cnt
        var = e2 - mean * mean
        s = gpre_ref[...] / jnp.sqrt(var + _EPS)
        stat[8:9] = s
        stat[9:10] = bpre_ref[...] - mean * s

    @pl.when(ph == 1)
    def _p1():
        m = mpt_ref[...].astype(jnp.float32)
        a1 = a1_fn()
        feat = jnp.maximum(a1 * stat[8:9] + stat[9:10], 0.0) * m
        pooled = jnp.max(feat.reshape(G, NPAD, H), axis=1)
        pc = jnp.dot(pooled, w1T_ref[H:2 * H, :],
                     preferred_element_type=jnp.float32)
        pc3 = jnp.broadcast_to(pc[:, None, :], (G, NPAD, H)).reshape(R, H)
        a2 = jnp.dot(feat, w1T_ref[0:H, :],
                     preferred_element_type=jnp.float32) + pc3
        accum(a2, m)
        a_out_ref[...] = a2.astype(jnp.bfloat16)

    @pl.when(jnp.logical_and(ph == 1, i == nb - 1))
    def _f1():
        finalize(g1_ref, b1_ref, 10)

    @pl.when(ph == 2)
    def _p2():
        m = mpt_ref[...].astype(jnp.float32)
        a2 = a_in_ref[...].astype(jnp.float32)
        h2 = jnp.maximum(a2 * stat[10:11] + stat[11:12], 0.0) * m
        a3 = jnp.dot(h2, w2T_ref[...], preferred_element_type=jnp.float32)
        accum(a3, m)
        z = jnp.where(m > 0.0, a3, _NEG)
        segmax[pl.ds(i * G, G), :] = jnp.max(z.reshape(G, NPAD, H), axis=1)

    @pl.when(jnp.logical_and(ph == 2, i == nb - 1))
    def _f2():
        finalize(g2_ref, b2_ref, 12)

    @pl.when(ph == 3)
    def _p3():
        sm = segmax[pl.ds(i * G, G), :]
        buf = jnp.maximum(sm * stat[12:13] + stat[13:14], 0.0)
        o1 = jnp.maximum(
            jnp.dot(buf, wo1T_ref[...], preferred_element_type=jnp.float32)
            + bo1_ref[...], 0.0)
        o = jnp.dot(o1, wo2T_ref[...],
                    preferred_element_type=jnp.float32) + bo2_ref[...]
        valid = sm[:, 0:1] > (0.5 * _NEG)
        out_ref[...] = o * valid.astype(jnp.float32)


def kernel(polylines, polylines_mask, W_pre, g_pre, b_pre,
           W1, g1, b1, W2, g2, b2, Wo1, bo1, Wo2, bo2):
    B, P, N, C = polylines.shape
    H = W_pre.shape[0]
    O = Wo2.shape[0]
    BP = B * P
    NPAD = ((N + 7) // 8) * 8
    G = 384
    NB = BP // G
    R = G * NPAD

    xp = jnp.pad(polylines.reshape(BP, N, C),
                 ((0, 0), (0, NPAD - N), (0, 0))).reshape(BP * NPAD, C)
    mpt = jnp.pad(polylines_mask.astype(jnp.bfloat16).reshape(BP, N),
                  ((0, 0), (0, NPAD - N))).reshape(BP * NPAD, 1)
    a_buf = jnp.zeros((BP * NPAD, H), jnp.bfloat16)

    row = lambda v: v.reshape(1, -1)

    def x_idx(ph, i):
        return (jnp.where(ph < 2, i, 0), 0)

    def pts_idx(ph, i):
        return (jnp.where(ph < 3, i, 0), 0)

    def a_in_idx(ph, i):
        return (jnp.where(ph == 2, i, 1), 0)

    def a_out_idx(ph, i):
        return (jnp.where(ph == 1, i, 0), 0)

    def poly_idx(ph, i):
        return (jnp.where(ph == 3, i, 0), 0)

    full = lambda shape: pl.BlockSpec(shape, lambda ph, i: (0, 0))

    body = functools.partial(_body, G=G, NPAD=NPAD, H=H, C9=C)

    out, _ = pl.pallas_call(
        body,
        grid=(4, NB),
        in_specs=[
            pl.BlockSpec((R, C), x_idx),
            pl.BlockSpec((R, 1), pts_idx),
            pl.BlockSpec((R, H), a_in_idx),
            full((C, H)), full((1, H)), full((1, H)),
            full((2 * H, H)), full((1, H)), full((1, H)),
            full((H, H)), full((1, H)), full((1, H)),
            full((H, H)), full((1, H)), full((H, O)), full((1, O)),
        ],
        out_specs=[
            pl.BlockSpec((G, O), poly_idx),
            pl.BlockSpec((R, H), a_out_idx),
        ],
        out_shape=[
            jax.ShapeDtypeStruct((BP, O), jnp.float32),
            jax.ShapeDtypeStruct((BP * NPAD, H), jnp.bfloat16),
        ],
        input_output_aliases={2: 1},
        scratch_shapes=[
            pltpu.VMEM((32, H), jnp.float32),
            pltpu.VMEM((BP, H), jnp.float32),
        ],
    )(xp, mpt, a_buf,
      W_pre.T, row(g_pre), row(b_pre),
      W1.T, row(g1), row(b1),
      W2.T, row(g2), row(b2),
      Wo1.T, row(bo1), Wo2.T, row(bo2))
    return out.reshape(B, P, O)


# speedup vs baseline: 1.2468x; 1.0257x over previous
import functools

import jax
import jax.numpy as jnp
from jax.experimental import pallas as pl
from jax.experimental.pallas import tpu as pltpu

_EPS = 1e-5
_NEG = -1e30


def _body(x_ref, mpt_ref, a_in_ref,
          wpreT_ref, gpre_ref, bpre_ref,
          w1T_ref, g1_ref, b1_ref,
          w2T_ref, g2_ref, b2_ref,
          wo1T_ref, bo1_ref, wo2T_ref, bo2_ref,
          out_ref, a_out_ref, stat, segmax,
          *, G, NPAD, H, C9):
    ph = pl.program_id(0)
    i = pl.program_id(1)
    nb = pl.num_programs(1)
    R = G * NPAD

    @pl.when(jnp.logical_and(ph == 0, i == 0))
    def _init():
        stat[...] = jnp.zeros_like(stat)

    def accum(a, m):
        am = a * m
        stat[0:1] += jnp.sum(am, axis=0, keepdims=True)
        stat[1:2] += jnp.sum(am * a, axis=0, keepdims=True)

    def finalize(g_ref, b_ref, srow):
        cnt = jnp.maximum(stat[14:15], 1.0)
        mean = stat[0:1] / cnt
        var = stat[1:2] / cnt - mean * mean
        s = g_ref[...] / jnp.sqrt(var + _EPS)
        t = b_ref[...] - mean * s
        stat[srow:srow + 1] = s
        stat[srow + 1:srow + 2] = t
        stat[0:2] = jnp.zeros((2, H), jnp.float32)

    def a1_fn():
        return jnp.dot(x_ref[...], wpreT_ref[...],
                       preferred_element_type=jnp.float32)

    @pl.when(ph == 0)
    def _p0():
        m = mpt_ref[...].astype(jnp.float32)
        x = x_ref[...]
        xm = x * m
        stat[16:16 + C9, 0:C9] += jax.lax.dot_general(
            xm, x, (((0,), (0,)), ((), ())),
            preferred_element_type=jnp.float32)
        stat[15:16, 0:C9] += jnp.sum(xm, axis=0, keepdims=True)
        stat[14:15] += jnp.sum(m)

    @pl.when(jnp.logical_and(ph == 0, i == nb - 1))
    def _f0():
        cnt = jnp.maximum(stat[14:15], 1.0)
        wpre = wpreT_ref[...]
        mean = jnp.dot(stat[15:16, 0:C9], wpre,
                       preferred_element_type=jnp.float32) / cnt
        t_w = jnp.dot(stat[16:16 + C9, 0:C9], wpre,
                      preferred_element_type=jnp.float32)
        e2 = jnp.sum(t_w * wpre, axis=0, keepdims=True) / cnt
        var = e2 - mean * mean
        s = gpre_ref[...] / jnp.sqrt(var + _EPS)
        stat[8:9] = s
        stat[9:10] = bpre_ref[...] - mean * s

    @pl.when(ph == 1)
    def _p1():
        m = mpt_ref[...].astype(jnp.float32)
        a1 = a1_fn()
        feat = jnp.maximum(a1 * stat[8:9] + stat[9:10], 0.0) * m
        pooled = jnp.max(feat.reshape(G, NPAD, H), axis=1)
        pc = jnp.dot(pooled, w1T_ref[H:2 * H, :],
                     preferred_element_type=jnp.float32)
        pc3 = jnp.broadcast_to(pc[:, None, :], (G, NPAD, H)).reshape(R, H)
        a2 = jnp.dot(feat, w1T_ref[0:H, :],
                     preferred_element_type=jnp.float32) + pc3
        accum(a2, m)
        a_out_ref[...] = a2.astype(jnp.bfloat16)

    @pl.when(jnp.logical_and(ph == 1, i == nb - 1))
    def _f1():
        finalize(g1_ref, b1_ref, 10)

    @pl.when(ph == 2)
    def _p2():
        m = mpt_ref[...].astype(jnp.float32)
        a2 = a_in_ref[...].astype(jnp.float32)
        h2 = jnp.maximum(a2 * stat[10:11] + stat[11:12], 0.0) * m
        a3 = jnp.dot(h2, w2T_ref[...], preferred_element_type=jnp.float32)
        accum(a3, m)
        z = jnp.where(m > 0.0, a3, _NEG)
        segmax[pl.ds(i * G, G), :] = jnp.max(z.reshape(G, NPAD, H), axis=1)

    @pl.when(jnp.logical_and(ph == 2, i == nb - 1))
    def _f2():
        finalize(g2_ref, b2_ref, 12)

    @pl.when(ph == 3)
    def _p3():
        sm = segmax[pl.ds(i * G, G), :]
        buf = jnp.maximum(sm * stat[12:13] + stat[13:14], 0.0)
        o1 = jnp.maximum(
            jnp.dot(buf, wo1T_ref[...], preferred_element_type=jnp.float32)
            + bo1_ref[...], 0.0)
        o = jnp.dot(o1, wo2T_ref[...],
                    preferred_element_type=jnp.float32) + bo2_ref[...]
        valid = sm[:, 0:1] > (0.5 * _NEG)
        out_ref[...] = o * valid.astype(jnp.float32)


def kernel(polylines, polylines_mask, W_pre, g_pre, b_pre,
           W1, g1, b1, W2, g2, b2, Wo1, bo1, Wo2, bo2):
    B, P, N, C = polylines.shape
    H = W_pre.shape[0]
    O = Wo2.shape[0]
    BP = B * P
    NPAD = ((N + 7) // 8) * 8
    G = 512
    NB = BP // G
    R = G * NPAD

    xp = jnp.pad(polylines.reshape(BP, N, C),
                 ((0, 0), (0, NPAD - N), (0, 0))).reshape(BP * NPAD, C)
    mpt = jnp.pad(polylines_mask.astype(jnp.bfloat16).reshape(BP, N),
                  ((0, 0), (0, NPAD - N))).reshape(BP * NPAD, 1)
    a_buf = jnp.zeros((BP * NPAD, H), jnp.bfloat16)

    row = lambda v: v.reshape(1, -1)

    def x_idx(ph, i):
        return (jnp.where(ph < 2, i, 0), 0)

    def pts_idx(ph, i):
        return (jnp.where(ph < 3, i, 0), 0)

    def a_in_idx(ph, i):
        return (jnp.where(ph == 2, i, 1), 0)

    def a_out_idx(ph, i):
        return (jnp.where(ph == 1, i, 0), 0)

    def poly_idx(ph, i):
        return (jnp.where(ph == 3, i, 0), 0)

    full = lambda shape: pl.BlockSpec(shape, lambda ph, i: (0, 0))

    body = functools.partial(_body, G=G, NPAD=NPAD, H=H, C9=C)

    out, _ = pl.pallas_call(
        body,
        grid=(4, NB),
        in_specs=[
            pl.BlockSpec((R, C), x_idx),
            pl.BlockSpec((R, 1), pts_idx),
            pl.BlockSpec((R, H), a_in_idx),
            full((C, H)), full((1, H)), full((1, H)),
            full((2 * H, H)), full((1, H)), full((1, H)),
            full((H, H)), full((1, H)), full((1, H)),
            full((H, H)), full((1, H)), full((H, O)), full((1, O)),
        ],
        out_specs=[
            pl.BlockSpec((G, O), poly_idx),
            pl.BlockSpec((R, H), a_out_idx),
        ],
        out_shape=[
            jax.ShapeDtypeStruct((BP, O), jnp.float32),
            jax.ShapeDtypeStruct((BP * NPAD, H), jnp.bfloat16),
        ],
        input_output_aliases={2: 1},
        scratch_shapes=[
            pltpu.VMEM((32, H), jnp.float32),
            pltpu.VMEM((BP, H), jnp.float32),
        ],
    )(xp, mpt, a_buf,
      W_pre.T, row(g_pre), row(b_pre),
      W1.T, row(g1), row(b1),
      W2.T, row(g2), row(b2),
      Wo1.T, row(bo1), Wo2.T, row(bo2))
    return out.reshape(B, P, O)


# speedup vs baseline: 1.2987x; 1.0417x over previous
import functools

import jax
import jax.numpy as jnp
from jax.experimental import pallas as pl
from jax.experimental.pallas import tpu as pltpu

_EPS = 1e-5
_NEG = -1e30


def _body(x_ref, mpt_ref,
          wpreT_ref, gpre_ref, bpre_ref,
          w1T_ref, g1_ref, b1_ref,
          w2T_ref, g2_ref, b2_ref,
          wo1T_ref, bo1_ref, wo2T_ref, bo2_ref,
          out_ref, ahbm_ref,
          stat, segmax, stg, lds, st_sem, ld_sem,
          *, G, NPAD, H, C9):
    ph = pl.program_id(0)
    i = pl.program_id(1)
    nb = pl.num_programs(1)
    R = G * NPAD
    slot = jax.lax.rem(i, 2)

    @pl.when(jnp.logical_and(ph == 0, i == 0))
    def _init():
        stat[...] = jnp.zeros_like(stat)

    def accum(a, m):
        am = a * m
        stat[0:1] += jnp.sum(am, axis=0, keepdims=True)
        stat[1:2] += jnp.sum(am * a, axis=0, keepdims=True)

    def finalize(g_ref, b_ref, srow):
        cnt = jnp.maximum(stat[14:15], 1.0)
        mean = stat[0:1] / cnt
        var = stat[1:2] / cnt - mean * mean
        s = g_ref[...] / jnp.sqrt(var + _EPS)
        t = b_ref[...] - mean * s
        stat[srow:srow + 1] = s
        stat[srow + 1:srow + 2] = t
        stat[0:2] = jnp.zeros((2, H), jnp.float32)

    @pl.when(ph == 0)
    def _p0():
        m = mpt_ref[...].astype(jnp.float32)
        x = x_ref[...]
        xm = x * m
        stat[16:16 + C9, 0:C9] += jax.lax.dot_general(
            xm, x, (((0,), (0,)), ((), ())),
            preferred_element_type=jnp.float32)
        stat[15:16, 0:C9] += jnp.sum(xm, axis=0, keepdims=True)
        stat[14:15] += jnp.sum(m)

    @pl.when(jnp.logical_and(ph == 0, i == nb - 1))
    def _f0():
        cnt = jnp.maximum(stat[14:15], 1.0)
        wpre = wpreT_ref[...]
        mean = jnp.dot(stat[15:16, 0:C9], wpre,
                       preferred_element_type=jnp.float32) / cnt
        t_w = jnp.dot(stat[16:16 + C9, 0:C9], wpre,
                      preferred_element_type=jnp.float32)
        e2 = jnp.sum(t_w * wpre, axis=0, keepdims=True) / cnt
        var = e2 - mean * mean
        s = gpre_ref[...] / jnp.sqrt(var + _EPS)
        stat[8:9] = s
        stat[9:10] = bpre_ref[...] - mean * s

    @pl.when(ph == 1)
    def _p1():
        m = mpt_ref[...].astype(jnp.float32)
        a1 = jnp.dot(x_ref[...], wpreT_ref[...],
                     preferred_element_type=jnp.float32)
        feat = jnp.maximum(a1 * stat[8:9] + stat[9:10], 0.0) * m
        pooled = jnp.max(feat.reshape(G, NPAD, H), axis=1)
        pc = jnp.dot(pooled, w1T_ref[H:2 * H, :],
                     preferred_element_type=jnp.float32)
        pc3 = jnp.broadcast_to(pc[:, None, :], (G, NPAD, H)).reshape(R, H)
        a2 = jnp.dot(feat, w1T_ref[0:H, :],
                     preferred_element_type=jnp.float32) + pc3
        accum(a2, m)
        @pl.when(i >= 2)
        def _():
            pltpu.make_async_copy(
                stg.at[slot], ahbm_ref.at[pl.ds(0, R), :],
                st_sem.at[slot]).wait()
        stg[slot] = a2.astype(jnp.bfloat16)
        pltpu.make_async_copy(
            stg.at[slot], ahbm_ref.at[pl.ds(i * R, R), :],
            st_sem.at[slot]).start()

    @pl.when(jnp.logical_and(ph == 1, i == nb - 1))
    def _f1():
        pltpu.make_async_copy(
            stg.at[0], ahbm_ref.at[pl.ds(0, R), :], st_sem.at[1 - slot]).wait()
        pltpu.make_async_copy(
            stg.at[0], ahbm_ref.at[pl.ds(0, R), :], st_sem.at[slot]).wait()
        finalize(g1_ref, b1_ref, 10)

    @pl.when(ph == 2)
    def _p2():
        @pl.when(i == 0)
        def _():
            pltpu.make_async_copy(
                ahbm_ref.at[pl.ds(0, R), :], lds.at[0], ld_sem.at[0]).start()
            pltpu.make_async_copy(
                ahbm_ref.at[pl.ds(R, R), :], lds.at[1], ld_sem.at[1]).start()
        pltpu.make_async_copy(lds.at[slot], lds.at[slot], ld_sem.at[slot]).wait()
        m = mpt_ref[...].astype(jnp.float32)
        a2 = lds[slot].astype(jnp.float32)
        h2 = jnp.maximum(a2 * stat[10:11] + stat[11:12], 0.0) * m
        a3 = jnp.dot(h2, w2T_ref[...], preferred_element_type=jnp.float32)
        accum(a3, m)
        z = jnp.where(m > 0.0, a3, _NEG)
        segmax[pl.ds(i * G, G), :] = jnp.max(z.reshape(G, NPAD, H), axis=1)
        @pl.when(i + 2 < nb)
        def _():
            pltpu.make_async_copy(
                ahbm_ref.at[pl.ds((i + 2) * R, R), :], lds.at[slot],
                ld_sem.at[slot]).start()

    @pl.when(jnp.logical_and(ph == 2, i == nb - 1))
    def _f2():
        finalize(g2_ref, b2_ref, 12)

    @pl.when(ph == 3)
    def _p3():
        sm = segmax[pl.ds(i * G, G), :]
        buf = jnp.maximum(sm * stat[12:13] + stat[13:14], 0.0)
        o1 = jnp.maximum(
            jnp.dot(buf, wo1T_ref[...], preferred_element_type=jnp.float32)
            + bo1_ref[...], 0.0)
        o = jnp.dot(o1, wo2T_ref[...],
                    preferred_element_type=jnp.float32) + bo2_ref[...]
        valid = sm[:, 0:1] > (0.5 * _NEG)
        out_ref[...] = o * valid.astype(jnp.float32)


def kernel(polylines, polylines_mask, W_pre, g_pre, b_pre,
           W1, g1, b1, W2, g2, b2, Wo1, bo1, Wo2, bo2):
    B, P, N, C = polylines.shape
    H = W_pre.shape[0]
    O = Wo2.shape[0]
    BP = B * P
    NPAD = ((N + 7) // 8) * 8
    G = 512
    NB = BP // G
    R = G * NPAD

    xp = jnp.pad(polylines.reshape(BP, N, C),
                 ((0, 0), (0, NPAD - N), (0, 0))).reshape(BP * NPAD, C)
    mpt = jnp.pad(polylines_mask.astype(jnp.bfloat16).reshape(BP, N),
                  ((0, 0), (0, NPAD - N))).reshape(BP * NPAD, 1)

    row = lambda v: v.reshape(1, -1)

    def x_idx(ph, i):
        return (jnp.where(ph < 2, i, 0), 0)

    def pts_idx(ph, i):
        return (jnp.where(ph < 3, i, 0), 0)

    def poly_idx(ph, i):
        return (jnp.where(ph == 3, i, 0), 0)

    full = lambda shape: pl.BlockSpec(shape, lambda ph, i: (0, 0))

    body = functools.partial(_body, G=G, NPAD=NPAD, H=H, C9=C)

    out, _ = pl.pallas_call(
        body,
        grid=(4, NB),
        in_specs=[
            pl.BlockSpec((R, C), x_idx),
            pl.BlockSpec((R, 1), pts_idx),
            full((C, H)), full((1, H)), full((1, H)),
            full((2 * H, H)), full((1, H)), full((1, H)),
            full((H, H)), full((1, H)), full((1, H)),
            full((H, H)), full((1, H)), full((H, O)), full((1, O)),
        ],
        out_specs=[
            pl.BlockSpec((G, O), poly_idx),
            pl.BlockSpec(memory_space=pltpu.MemorySpace.HBM),
        ],
        out_shape=[
            jax.ShapeDtypeStruct((BP, O), jnp.float32),
            jax.ShapeDtypeStruct((BP * NPAD, H), jnp.bfloat16),
        ],
        scratch_shapes=[
            pltpu.VMEM((32, H), jnp.float32),
            pltpu.VMEM((BP, H), jnp.float32),
            pltpu.VMEM((2, R, H), jnp.bfloat16),
            pltpu.VMEM((2, R, H), jnp.bfloat16),
            pltpu.SemaphoreType.DMA((2,)),
            pltpu.SemaphoreType.DMA((2,)),
        ],
    )(xp, mpt,
      W_pre.T, row(g_pre), row(b_pre),
      W1.T, row(g1), row(b1),
      W2.T, row(g2), row(b2),
      Wo1.T, row(bo1), Wo2.T, row(bo2))
    return out.reshape(B, P, O)


# speedup vs baseline: 1.3146x; 1.0122x over previous
import functools

import jax
import jax.numpy as jnp
from jax.experimental import pallas as pl
from jax.experimental.pallas import tpu as pltpu

_EPS = 1e-5
_NEG = -1e30


def _body(xa_ref,
          wpreT_ref, gpre_ref, bpre_ref,
          w1T_ref, g1_ref, b1_ref,
          w2T_ref, g2_ref, b2_ref,
          wo1T_ref, bo1_ref, wo2T_ref, bo2_ref,
          out_ref, ahbm_ref,
          stat, segmax, stg, lds, st_sem, ld_sem,
          *, G, NPAD, H, C9):
    ph = pl.program_id(0)
    i = pl.program_id(1)
    nb = pl.num_programs(1)
    R = G * NPAD
    slot = jax.lax.rem(i, 2)

    @pl.when(jnp.logical_and(ph == 0, i == 0))
    def _init():
        stat[...] = jnp.zeros_like(stat)

    def finalize(g_ref, b_ref, srow):
        cnt = jnp.maximum(stat[14:15], 1.0)
        mean = stat[0:1] / cnt
        var = stat[1:2] / cnt - mean * mean
        s = g_ref[...] / jnp.sqrt(var + _EPS)
        t = b_ref[...] - mean * s
        stat[srow:srow + 1] = s
        stat[srow + 1:srow + 2] = t
        stat[0:2] = jnp.zeros((2, H), jnp.float32)

    @pl.when(ph == 0)
    def _p0():
        xm = xa_ref[:, 0:C9]
        stat[16:16 + C9, 0:C9] += jax.lax.dot_general(
            xm, xm, (((0,), (0,)), ((), ())),
            preferred_element_type=jnp.float32)
        stat[15:16, 0:C9] += jnp.sum(xm, axis=0, keepdims=True)
        stat[14:15] += jnp.sum(xa_ref[:, C9:C9 + 1])

    @pl.when(jnp.logical_and(ph == 0, i == nb - 1))
    def _f0():
        cnt = jnp.maximum(stat[14:15], 1.0)
        wpre = wpreT_ref[...]
        mean = jnp.dot(stat[15:16, 0:C9], wpre,
                       preferred_element_type=jnp.float32) / cnt
        t_w = jnp.dot(stat[16:16 + C9, 0:C9], wpre,
                      preferred_element_type=jnp.float32)
        e2 = jnp.sum(t_w * wpre, axis=0, keepdims=True) / cnt
        var = e2 - mean * mean
        s = gpre_ref[...] / jnp.sqrt(var + _EPS)
        stat[8:9] = s
        stat[9:10] = bpre_ref[...] - mean * s

    @pl.when(ph == 1)
    def _p1():
        xa = xa_ref[...]
        m = xa[:, C9:C9 + 1]
        wfold = jnp.concatenate([wpreT_ref[...] * stat[8:9], stat[9:10]],
                                axis=0)
        feat = jnp.maximum(
            jnp.dot(xa, wfold, preferred_element_type=jnp.float32), 0.0)
        pooled = jnp.max(feat.reshape(G, NPAD, H), axis=1)
        pc = jnp.dot(pooled, w1T_ref[H:2 * H, :],
                     preferred_element_type=jnp.float32)
        pc3 = jnp.broadcast_to(pc[:, None, :], (G, NPAD, H)).reshape(R, H)
        a2 = jnp.dot(feat, w1T_ref[0:H, :],
                     preferred_element_type=jnp.float32) + pc3
        a2m = a2 * m
        stat[0:1] += jnp.sum(a2m, axis=0, keepdims=True)
        stat[1:2] += jnp.sum(a2m * a2m, axis=0, keepdims=True)
        @pl.when(i >= 2)
        def _():
            pltpu.make_async_copy(
                stg.at[slot], ahbm_ref.at[pl.ds(0, R), :],
                st_sem.at[slot]).wait()
        stg[slot, :, 0:H] = a2m.astype(jnp.bfloat16)
        stg[slot, :, H:H + 1] = m.astype(jnp.bfloat16)
        pltpu.make_async_copy(
            stg.at[slot], ahbm_ref.at[pl.ds(i * R, R), :],
            st_sem.at[slot]).start()

    @pl.when(jnp.logical_and(ph == 1, i == nb - 1))
    def _f1():
        pltpu.make_async_copy(
            stg.at[0], ahbm_ref.at[pl.ds(0, R), :], st_sem.at[1 - slot]).wait()
        pltpu.make_async_copy(
            stg.at[0], ahbm_ref.at[pl.ds(0, R), :], st_sem.at[slot]).wait()
        finalize(g1_ref, b1_ref, 10)

    @pl.when(ph == 2)
    def _p2():
        @pl.when(i == 0)
        def _():
            pltpu.make_async_copy(
                ahbm_ref.at[pl.ds(0, R), :], lds.at[0], ld_sem.at[0]).start()
            pltpu.make_async_copy(
                ahbm_ref.at[pl.ds(R, R), :], lds.at[1], ld_sem.at[1]).start()
        pltpu.make_async_copy(lds.at[slot], lds.at[slot], ld_sem.at[slot]).wait()
        y = lds[slot].astype(jnp.float32)
        a2m = y[:, 0:H]
        m = y[:, H:H + 1]
        h2 = jnp.maximum(a2m * stat[10:11] + m * stat[11:12], 0.0)
        a3 = jnp.dot(h2, w2T_ref[...], preferred_element_type=jnp.float32)
        stat[0:1] += jnp.sum(a3, axis=0, keepdims=True)
        stat[1:2] += jnp.sum(a3 * a3, axis=0, keepdims=True)
        z = jnp.where(m > 0.0, a3, _NEG)
        segmax[pl.ds(i * G, G), :] = jnp.max(z.reshape(G, NPAD, H), axis=1)
        @pl.when(i + 2 < nb)
        def _():
            pltpu.make_async_copy(
                ahbm_ref.at[pl.ds((i + 2) * R, R), :], lds.at[slot],
                ld_sem.at[slot]).start()

    @pl.when(jnp.logical_and(ph == 2, i == nb - 1))
    def _f2():
        finalize(g2_ref, b2_ref, 12)

    @pl.when(ph == 3)
    def _p3():
        sm = segmax[pl.ds(i * G, G), :]
        buf = jnp.maximum(sm * stat[12:13] + stat[13:14], 0.0)
        o1 = jnp.maximum(
            jnp.dot(buf, wo1T_ref[...], preferred_element_type=jnp.float32)
            + bo1_ref[...], 0.0)
        o = jnp.dot(o1, wo2T_ref[...],
                    preferred_element_type=jnp.float32) + bo2_ref[...]
        valid = sm[:, 0:1] > (0.5 * _NEG)
        out_ref[...] = o * valid.astype(jnp.float32)


def kernel(polylines, polylines_mask, W_pre, g_pre, b_pre,
           W1, g1, b1, W2, g2, b2, Wo1, bo1, Wo2, bo2):
    B, P, N, C = polylines.shape
    H = W_pre.shape[0]
    O = Wo2.shape[0]
    BP = B * P
    NPAD = ((N + 7) // 8) * 8
    G = 512
    NB = BP // G
    R = G * NPAD

    mf = polylines_mask.astype(jnp.float32)
    xa = jnp.concatenate([polylines * mf[..., None], mf[..., None]], axis=-1)
    xa = jnp.pad(xa.reshape(BP, N, C + 1),
                 ((0, 0), (0, NPAD - N), (0, 0))).reshape(BP * NPAD, C + 1)

    row = lambda v: v.reshape(1, -1)

    def x_idx(ph, i):
        return (jnp.where(ph < 2, i, 0), 0)

    def poly_idx(ph, i):
        return (jnp.where(ph == 3, i, 0), 0)

    full = lambda shape: pl.BlockSpec(shape, lambda ph, i: (0, 0))

    body = functools.partial(_body, G=G, NPAD=NPAD, H=H, C9=C)

    out, _ = pl.pallas_call(
        body,
        grid=(4, NB),
        in_specs=[
            pl.BlockSpec((R, C + 1), x_idx),
            full((C, H)), full((1, H)), full((1, H)),
            full((2 * H, H)), full((1, H)), full((1, H)),
            full((H, H)), full((1, H)), full((1, H)),
            full((H, H)), full((1, H)), full((H, O)), full((1, O)),
        ],
        out_specs=[
            pl.BlockSpec((G, O), poly_idx),
            pl.BlockSpec(memory_space=pltpu.MemorySpace.HBM),
        ],
        out_shape=[
            jax.ShapeDtypeStruct((BP, O), jnp.float32),
            jax.ShapeDtypeStruct((BP * NPAD, H + 1), jnp.bfloat16),
        ],
        scratch_shapes=[
            pltpu.VMEM((32, H), jnp.float32),
            pltpu.VMEM((BP, H), jnp.float32),
            pltpu.VMEM((2, R, H + 1), jnp.bfloat16),
            pltpu.VMEM((2, R, H + 1), jnp.bfloat16),
            pltpu.SemaphoreType.DMA((2,)),
            pltpu.SemaphoreType.DMA((2,)),
        ],
    )(xa,
      W_pre.T, row(g_pre), row(b_pre),
      W1.T, row(g1), row(b1),
      W2.T, row(g2), row(b2),
      Wo1.T, row(bo1), Wo2.T, row(bo2))
    return out.reshape(B, P, O)
